# baseline (speedup 1.0000x reference)
import functools

import jax
import jax.numpy as jnp
from jax import lax
from jax.experimental import pallas as pl
from jax.experimental.pallas import tpu as pltpu
from jax.experimental.pallas import tpu_sc as plsc

B, T, K = 1024, 50, 20
EMB, HID = 64, 128

NC, NS = 2, 16
NW = NC * NS
BPW = B // NW
RPW = BPW * K
CH = 128
NCHUNK = RPW // CH

@functools.cache
def _make_gather_meanpool():
    mesh = plsc.VectorSubcoreMesh(core_axis_name="c", subcore_axis_name="s")
    ncol = EMB // 16

    @functools.partial(
        pl.kernel,
        out_type=jax.ShapeDtypeStruct((B, EMB), jnp.float32),
        mesh=mesh,
        compiler_params=pltpu.CompilerParams(use_tc_tiling_on_sc=False),
        scratch_types=[
            pltpu.VMEM((NCHUNK, CH), jnp.int32),
            pltpu.VMEM((RPW, EMB), jnp.float32),
            pltpu.VMEM((BPW, EMB), jnp.float32),
            pltpu.SemaphoreType.DMA,
        ],
    )
    def _gather_meanpool(table_hbm, idx_hbm, out_hbm, idx_v, rows_v, acc_v,
                         sem):
        wid = lax.axis_index("s") * NC + lax.axis_index("c")
        pltpu.sync_copy(idx_hbm.at[wid], idx_v)
        cps = [
            pltpu.async_copy(
                table_hbm.at[idx_v.at[j]], rows_v.at[pl.ds(j * CH, CH)], sem
            )
            for j in range(NCHUNK)
        ]
        for cp in cps:
            cp.wait()

        def body(lb, carry):
            base = lb * K
            for c in range(ncol):
                col = pl.ds(c * 16, 16)
                acc = rows_v[base, col]
                for k in range(1, K):
                    acc = acc + rows_v[base + k, col]
                acc_v[lb, col] = acc * (1.0 / K)
            return carry

        lax.fori_loop(0, BPW, body, 0)
        pltpu.sync_copy(acc_v, out_hbm.at[pl.ds(wid * BPW, BPW)])

    return _gather_meanpool


def _dense_body(xm_ref, w0_ref, b0_ref, w1_ref, b1_ref, wfc_ref, bfc_ref,
                out_ref):
    xm = xm_ref[...]
    g0 = jnp.dot(xm, w0_ref[...], preferred_element_type=jnp.float32,
                 precision=lax.Precision.HIGHEST)
    g0 = g0 + b0_ref[...]
    c0 = jax.nn.sigmoid(g0[:, 0:HID]) * jnp.tanh(g0[:, HID:2 * HID])
    h0 = jax.nn.sigmoid(g0[:, 2 * HID:3 * HID]) * jnp.tanh(c0)
    g1 = jnp.dot(h0, w1_ref[...], preferred_element_type=jnp.float32,
                 precision=lax.Precision.HIGHEST)
    g1 = g1 + b1_ref[...]
    c1 = jax.nn.sigmoid(g1[:, 0:HID]) * jnp.tanh(g1[:, HID:2 * HID])
    h1 = jax.nn.sigmoid(g1[:, 2 * HID:3 * HID]) * jnp.tanh(c1)
    out_ref[...] = (
        jnp.sum(h1 * wfc_ref[...], axis=1, keepdims=True) + bfc_ref[...]
    )


_dense_call = pl.pallas_call(
    _dense_body,
    out_shape=jax.ShapeDtypeStruct((B, 1), jnp.float32),
)


def _prune_gates(W, b_ih, b_hh):
    Wp = jnp.concatenate([W[0:HID], W[2 * HID:4 * HID]], axis=0)
    b = b_ih + b_hh
    bp = jnp.concatenate([b[0:HID], b[2 * HID:4 * HID]])
    return Wp.T, bp[None, :]


@functools.cache
def _make_trivial_sc():
    mesh = plsc.VectorSubcoreMesh(core_axis_name="c", subcore_axis_name="s",
                                  num_cores=1)

    @functools.partial(
        pl.kernel,
        out_type=jax.ShapeDtypeStruct((NW, 16), jnp.int32),
        mesh=mesh,
        scratch_types=[
            pltpu.VMEM((16,), jnp.int32),
        ],
    )
    def _trivial(idx_hbm, out_hbm, acc_v):
        wid = lax.axis_index("s") * NC + lax.axis_index("c")
        pltpu.sync_copy(idx_hbm.at[wid, 0], acc_v)
        pltpu.sync_copy(acc_v, out_hbm.at[wid])

    return _trivial


def kernel(x, len_batch, table, W_ih0, W_hh0, b_ih0, b_hh0,
           W_ih1, W_hh1, b_ih1, b_hh1, W_fc, b_fc):
    idx = x[:, 0, :].reshape(NW, NCHUNK, CH)
    xm = _make_gather_meanpool()(table, idx)
    w0, b0 = _prune_gates(W_ih0, b_ih0, b_hh0)
    w1, b1 = _prune_gates(W_ih1, b_ih1, b_hh1)
    out = _dense_call(xm, w0, b0, w1, b1, W_fc, b_fc[None, :])
    return (out.reshape(B, 1, 1), len_batch)
    w0, b0 = _prune_gates(W_ih0, b_ih0, b_hh0)
    w1, b1 = _prune_gates(W_ih1, b_ih1, b_hh1)
    out = _dense_call(xm, w0, b0, w1, b1, W_fc, b_fc[None, :])
    return (out.reshape(B, 1, 1), len_batch)


# speedup vs baseline: 1.0030x; 1.0030x over previous
import functools

import jax
import jax.numpy as jnp
from jax import lax
from jax.experimental import pallas as pl
from jax.experimental.pallas import tpu as pltpu
from jax.experimental.pallas import tpu_sc as plsc

B, T, K = 1024, 50, 20
VOCAB, EMB, HID = 1000000, 64, 128

NC, NS = 2, 16
NW = NC * NS
BPW = B // NW
RPW = BPW * K
CH = 128
NCHUNK = RPW // CH

@functools.cache
def _make_gather_meanpool():
    mesh = plsc.VectorSubcoreMesh(core_axis_name="c", subcore_axis_name="s")
    ncol = EMB // 16
    ngrp = BPW // 4
    nvec = 4 * K // 16

    @functools.partial(
        pl.kernel,
        out_type=jax.ShapeDtypeStruct((B, EMB), jnp.float32),
        mesh=mesh,
        scratch_types=[
            pltpu.VMEM((RPW // 16, 16), jnp.int32),
            pltpu.VMEM((NCHUNK, CH), jnp.int32),
            pltpu.VMEM((RPW, 2 * EMB), jnp.float32),
            pltpu.VMEM((BPW, EMB), jnp.float32),
            pltpu.SemaphoreType.DMA,
        ],
    )
    def _gather_meanpool(table_hbm, idx_hbm, out_hbm, idx_v, idxp_v, rows_v,
                         acc_v, sem):
        wid = lax.axis_index("s") * NC + lax.axis_index("c")
        pltpu.sync_copy(idx_hbm.at[wid], idx_v)
        for t in range(RPW // 16):
            idxp_v[t // 8, pl.ds((t % 8) * 16, 16)] = idx_v[t] >> 1
        cps = [
            pltpu.async_copy(
                table_hbm.at[idxp_v.at[j]], rows_v.at[pl.ds(j * CH, CH)], sem
            )
            for j in range(NCHUNK)
        ]
        for cp in cps:
            cp.wait()

        def load_vecs(h):
            return [idx_v[nvec * h + m] for m in range(nvec)]

        def get_i(vecs, j, k):
            p = K * j + k
            return vecs[p // 16][p % 16]

        def accum(vecs, j, lb):
            accs = None
            for k in range(K):
                i = get_i(vecs, j, k)
                off = (i & 1) * EMB
                vals = [rows_v[lb * K + k, pl.ds(off + c * 16, 16)]
                        for c in range(ncol)]
                accs = vals if accs is None else (
                    [a + v for a, v in zip(accs, vals)])
            for c in range(ncol):
                acc_v[lb, pl.ds(c * 16, 16)] = accs[c] * (1.0 / K)

        def loop_body(h, carry):
            vecs = load_vecs(h)
            for j in range(4):
                accum(vecs, j, 4 * h + j)
            return carry

        lax.fori_loop(0, ngrp, loop_body, 0)
        pltpu.sync_copy(acc_v, out_hbm.at[pl.ds(wid * BPW, BPW)])

    return _gather_meanpool


def _dense_body(xm_ref, w0_ref, b0_ref, w1_ref, b1_ref, wfc_ref, bfc_ref,
                out_ref):
    xm = xm_ref[...]
    g0 = jnp.dot(xm, w0_ref[...], preferred_element_type=jnp.float32,
                 precision=lax.Precision.HIGHEST)
    g0 = g0 + b0_ref[...]
    c0 = jax.nn.sigmoid(g0[:, 0:HID]) * jnp.tanh(g0[:, HID:2 * HID])
    h0 = jax.nn.sigmoid(g0[:, 2 * HID:3 * HID]) * jnp.tanh(c0)
    g1 = jnp.dot(h0, w1_ref[...], preferred_element_type=jnp.float32,
                 precision=lax.Precision.HIGHEST)
    g1 = g1 + b1_ref[...]
    c1 = jax.nn.sigmoid(g1[:, 0:HID]) * jnp.tanh(g1[:, HID:2 * HID])
    h1 = jax.nn.sigmoid(g1[:, 2 * HID:3 * HID]) * jnp.tanh(c1)
    out_ref[...] = (
        jnp.sum(h1 * wfc_ref[...], axis=1, keepdims=True) + bfc_ref[...]
    )


_dense_call = pl.pallas_call(
    _dense_body,
    out_shape=jax.ShapeDtypeStruct((B, 1), jnp.float32),
)


def _prune_gates(W, b_ih, b_hh):
    Wp = jnp.concatenate([W[0:HID], W[2 * HID:4 * HID]], axis=0)
    b = b_ih + b_hh
    bp = jnp.concatenate([b[0:HID], b[2 * HID:4 * HID]])
    return Wp.T, bp[None, :]


@functools.cache
def _make_trivial_sc():
    mesh = plsc.VectorSubcoreMesh(core_axis_name="c", subcore_axis_name="s",
                                  num_cores=1)

    @functools.partial(
        pl.kernel,
        out_type=jax.ShapeDtypeStruct((NW, 16), jnp.int32),
        mesh=mesh,
        scratch_types=[
            pltpu.VMEM((16,), jnp.int32),
        ],
    )
    def _trivial(idx_hbm, out_hbm, acc_v):
        wid = lax.axis_index("s") * NC + lax.axis_index("c")
        pltpu.sync_copy(idx_hbm.at[wid, 0], acc_v)
        pltpu.sync_copy(acc_v, out_hbm.at[wid])

    return _trivial


def kernel(x, len_batch, table, W_ih0, W_hh0, b_ih0, b_hh0,
           W_ih1, W_hh1, b_ih1, b_hh1, W_fc, b_fc):
    idx = x[:, 0, :].reshape(NW, RPW // 16, 16)
    table2 = table.reshape(VOCAB // 2, 2 * EMB)
    xm = _make_gather_meanpool()(table2, idx)
    w0, b0 = _prune_gates(W_ih0, b_ih0, b_hh0)
    w1, b1 = _prune_gates(W_ih1, b_ih1, b_hh1)
    out = _dense_call(xm, w0, b0, w1, b1, W_fc, b_fc[None, :])
    return (out.reshape(B, 1, 1), len_batch)
    w0, b0 = _prune_gates(W_ih0, b_ih0, b_hh0)
    w1, b1 = _prune_gates(W_ih1, b_ih1, b_hh1)
    out = _dense_call(xm, w0, b0, w1, b1, W_fc, b_fc[None, :])
    return (out.reshape(B, 1, 1), len_batch)


# speedup vs baseline: 1.5849x; 1.5802x over previous
import functools

import jax
import jax.numpy as jnp
from jax import lax
from jax.experimental import pallas as pl
from jax.experimental.pallas import tpu as pltpu
from jax.experimental.pallas import tpu_sc as plsc

B, T, K = 1024, 50, 20
EMB, HID = 64, 128

NC, NS = 2, 16
NW = NC * NS
BPW = B // NW
RPW = BPW * K
CH = 128
NCHUNK = RPW // CH

@functools.cache
def _make_gather_meanpool():
    mesh = plsc.VectorSubcoreMesh(core_axis_name="c", subcore_axis_name="s")
    ncol = EMB // 16
    nring = 4
    groups = BPW // nring
    nvec = nring * K // 16

    @functools.partial(
        pl.kernel,
        out_type=jax.ShapeDtypeStruct((B, EMB), jnp.float32),
        mesh=mesh,
        scratch_types=[
            pltpu.VMEM((RPW // 16, 16), jnp.int32),
            pltpu.VMEM((nring, K, 8, EMB), jnp.float32),
            pltpu.VMEM((BPW, EMB), jnp.float32),
            pltpu.SemaphoreType.DMA((nring,)),
        ],
    )
    def _gather_meanpool(table_hbm, idx_hbm, out_hbm, idx_v, bufs, acc_v,
                         sems):
        wid = lax.axis_index("s") * NC + lax.axis_index("c")
        pltpu.sync_copy(idx_hbm.at[wid], idx_v)

        def load_vecs(h):
            return [idx_v[nvec * h + m] for m in range(nvec)]

        def get_i(vecs, j, k):
            p = K * j + k
            return vecs[p // 16][p % 16]

        def issue(vecs, j):
            for k in range(K):
                i = get_i(vecs, j, k)
                base8 = pl.multiple_of((i >> 3) << 3, 8)
                pltpu.async_copy(
                    table_hbm.at[pl.ds(base8, 8)],
                    bufs.at[j, k],
                    sems.at[j],
                )

        def drain_accum(vecs, j, lb):
            for k in range(K):
                pltpu.make_async_copy(
                    table_hbm.at[pl.ds(0, 8)], bufs.at[j, k], sems.at[j]
                ).wait()
            accs = None
            for k in range(K):
                sub = get_i(vecs, j, k) & 7
                vals = [bufs[j, k, sub, pl.ds(c * 16, 16)]
                        for c in range(ncol)]
                accs = vals if accs is None else (
                    [a + v for a, v in zip(accs, vals)])
            for c in range(ncol):
                acc_v[lb, pl.ds(c * 16, 16)] = accs[c] * (1.0 / K)

        vecs0 = load_vecs(0)
        for j in range(nring):
            issue(vecs0, j)

        def loop_body(h, carry):
            vecs = load_vecs(h)
            nxt = load_vecs(h + 1)
            for j in range(nring):
                drain_accum(vecs, j, nring * h + j)
                issue(nxt, j)
            return carry

        lax.fori_loop(0, groups - 1, loop_body, 0)
        vecs_last = load_vecs(groups - 1)
        for j in range(nring):
            drain_accum(vecs_last, j, nring * (groups - 1) + j)
        pltpu.sync_copy(acc_v, out_hbm.at[pl.ds(wid * BPW, BPW)])

    return _gather_meanpool


def _dense_body(xm_ref, w0_ref, b0_ref, w1_ref, b1_ref, wfc_ref, bfc_ref,
                out_ref):
    xm = xm_ref[...]
    g0 = jnp.dot(xm, w0_ref[...], preferred_element_type=jnp.float32,
                 precision=lax.Precision.HIGHEST)
    g0 = g0 + b0_ref[...]
    c0 = jax.nn.sigmoid(g0[:, 0:HID]) * jnp.tanh(g0[:, HID:2 * HID])
    h0 = jax.nn.sigmoid(g0[:, 2 * HID:3 * HID]) * jnp.tanh(c0)
    g1 = jnp.dot(h0, w1_ref[...], preferred_element_type=jnp.float32,
                 precision=lax.Precision.HIGHEST)
    g1 = g1 + b1_ref[...]
    c1 = jax.nn.sigmoid(g1[:, 0:HID]) * jnp.tanh(g1[:, HID:2 * HID])
    h1 = jax.nn.sigmoid(g1[:, 2 * HID:3 * HID]) * jnp.tanh(c1)
    out_ref[...] = (
        jnp.sum(h1 * wfc_ref[...], axis=1, keepdims=True) + bfc_ref[...]
    )


_dense_call = pl.pallas_call(
    _dense_body,
    out_shape=jax.ShapeDtypeStruct((B, 1), jnp.float32),
)


def _prune_gates(W, b_ih, b_hh):
    Wp = jnp.concatenate([W[0:HID], W[2 * HID:4 * HID]], axis=0)
    b = b_ih + b_hh
    bp = jnp.concatenate([b[0:HID], b[2 * HID:4 * HID]])
    return Wp.T, bp[None, :]


def kernel(x, len_batch, table, W_ih0, W_hh0, b_ih0, b_hh0,
           W_ih1, W_hh1, b_ih1, b_hh1, W_fc, b_fc):
    idx = x[:, 0, :].reshape(NW, RPW // 16, 16)
    xm = _make_gather_meanpool()(table, idx)
    w0, b0 = _prune_gates(W_ih0, b_ih0, b_hh0)
    w1, b1 = _prune_gates(W_ih1, b_ih1, b_hh1)
    out = _dense_call(xm, w0, b0, w1, b1, W_fc, b_fc[None, :])
    return (out.reshape(B, 1, 1), len_batch)
